# baseline (speedup 1.0000x reference)
import jax
import jax.numpy as jnp
from jax import lax
from jax.experimental import pallas as pl
from jax.experimental.pallas import tpu as pltpu

_ALPHA = 0.2


def _elu(v):
    return jnp.where(v > 0, v, jnp.exp(jnp.minimum(v, 0.0)) - 1.0)


def _stgat_kernel(nhead, nhid, npred,
                  x_ref, adj_ref, wall_ref, a1_ref, a2_ref, a2t_ref,
                  wout_ref, a1ot_ref, a2ot_ref, out_ref,
                  adj_s, va_s, r_s, g1t_s, fb1_s, h_s,
                  vap_s, rp_s, g1pt_s, fbp_s):
    ph = pl.program_id(1)
    bf16 = jnp.bfloat16

    @pl.when(ph == 0)
    def _layer1():
        adj_s[...] = adj_ref[0].astype(bf16)
        wh = jnp.dot(x_ref[0], wall_ref[...],
                     preferred_element_type=jnp.float32)
        wh1 = jnp.dot(wh, a1_ref[...],
                      preferred_element_type=jnp.float32)
        wh2 = jnp.dot(wh, a2_ref[...],
                      preferred_element_type=jnp.float32)
        wh2t = lax.dot_general(
            a2t_ref[...], wh, (((1,), (1,)), ((), ())),
            preferred_element_type=jnp.float32)
        r_s[...] = jnp.exp((_ALPHA - 1.0) * wh1).astype(bf16)
        g1t_s[...] = jnp.exp((1.0 - _ALPHA) * wh2t).astype(bf16)
        f2 = jnp.exp(_ALPHA * wh2)
        for h in range(nhead):
            whh = wh[:, h * nhid:(h + 1) * nhid]
            f2c = f2[:, h:h + 1]
            va_s[h, :, :nhid] = (whh * f2c).astype(bf16)
            va_s[h, :, nhid:nhid + 1] = f2c.astype(bf16)
            fb1_s[h:h + 1, :] = jnp.mean(whh, axis=0, keepdims=True)

        adjb = adj_s[...]
        rb = r_s[...]
        outs = []
        for h in range(nhead):
            m = jnp.maximum(g1t_s[h:h + 1, :], rb[:, h:h + 1])
            w = adjb * m
            na = jnp.dot(w, va_s[h], preferred_element_type=jnp.float32)
            num = na[:, :nhid]
            den = na[:, nhid:nhid + 1]
            rden = jnp.where(den > 0, 1.0 / den, 0.0)
            hp = jnp.where(den > 0, num * rden, fb1_s[h:h + 1, :])
            outs.append(_elu(hp))
        h_s[...] = jnp.concatenate(outs, axis=1)

    @pl.when(ph == 1)
    def _layer2():
        whp = jnp.dot(h_s[...], wout_ref[...],
                      preferred_element_type=jnp.float32)
        wh1p = lax.dot_general(
            whp, a1ot_ref[...], (((1,), (1,)), ((), ())),
            preferred_element_type=jnp.float32)
        wh2pt = lax.dot_general(
            a2ot_ref[...], whp, (((1,), (1,)), ((), ())),
            preferred_element_type=jnp.float32)
        wh2p = lax.dot_general(
            whp, a2ot_ref[...], (((1,), (1,)), ((), ())),
            preferred_element_type=jnp.float32)
        rp_s[...] = jnp.exp((_ALPHA - 1.0) * wh1p).astype(bf16)
        g1pt_s[...] = jnp.exp((1.0 - _ALPHA) * wh2pt).astype(bf16)
        f2pc = jnp.exp(_ALPHA * wh2p)
        vap_s[:, :npred] = (whp * f2pc).astype(bf16)
        vap_s[:, npred:npred + 1] = f2pc.astype(bf16)
        fbp_s[...] = jnp.mean(whp, axis=0, keepdims=True)

        adjb = adj_s[...]
        m = jnp.maximum(g1pt_s[...], rp_s[...])
        w = adjb * m
        na = jnp.dot(w, vap_s[...], preferred_element_type=jnp.float32)
        num = na[:, :npred]
        den = na[:, npred:npred + 1]
        rden = jnp.where(den > 0, 1.0 / den, 0.0)
        hp = jnp.where(den > 0, num * rden, fbp_s[...])
        out_ref[0] = _elu(hp)


def kernel(x, adj, Ws, a_s, W_out, a_out):
    B, N, F = x.shape
    nhead, _, nhid = Ws.shape
    npred = W_out.shape[1]
    H = nhead * nhid

    wall = jnp.transpose(Ws, (1, 0, 2)).reshape(F, H)
    eye = jnp.eye(nhead, dtype=x.dtype)
    a1 = a_s[:, :nhid, 0]
    a2 = a_s[:, nhid:, 0]
    a1bd = (eye[:, :, None] * a1[None, :, :]).reshape(nhead, H).T
    a2bd = (eye[:, :, None] * a2[None, :, :]).reshape(nhead, H).T
    a2t = a2bd.T
    a1ot = a_out[:npred].T
    a2ot = a_out[npred:].T

    grid = (B, 2)

    def _idx_const(b, ph):
        return (0, 0)

    out = pl.pallas_call(
        lambda *refs: _stgat_kernel(nhead, nhid, npred, *refs),
        grid=grid,
        in_specs=[
            pl.BlockSpec((1, N, F), lambda b, ph: (b, 0, 0)),
            pl.BlockSpec((1, N, N), lambda b, ph: (b, 0, 0)),
            pl.BlockSpec((F, H), _idx_const),
            pl.BlockSpec((H, nhead), _idx_const),
            pl.BlockSpec((H, nhead), _idx_const),
            pl.BlockSpec((nhead, H), _idx_const),
            pl.BlockSpec((H, npred), _idx_const),
            pl.BlockSpec((1, npred), _idx_const),
            pl.BlockSpec((1, npred), _idx_const),
        ],
        out_specs=pl.BlockSpec((1, N, npred), lambda b, ph: (b, 0, 0)),
        out_shape=jax.ShapeDtypeStruct((B, N, npred), jnp.float32),
        scratch_shapes=[
            pltpu.VMEM((N, N), jnp.bfloat16),
            pltpu.VMEM((nhead, N, nhid + 1), jnp.bfloat16),
            pltpu.VMEM((N, nhead), jnp.bfloat16),
            pltpu.VMEM((nhead, N), jnp.bfloat16),
            pltpu.VMEM((nhead, nhid), jnp.float32),
            pltpu.VMEM((N, H), jnp.float32),
            pltpu.VMEM((N, npred + 1), jnp.bfloat16),
            pltpu.VMEM((N, 1), jnp.bfloat16),
            pltpu.VMEM((1, N), jnp.bfloat16),
            pltpu.VMEM((1, npred), jnp.float32),
        ],
    )(x, adj, wall, a1bd, a2bd, a2t, W_out, a1ot, a2ot)
    return out.reshape(B, N * npred)


# speedup vs baseline: 1.6870x; 1.6870x over previous
import jax
import jax.numpy as jnp
from jax import lax
from jax.experimental import pallas as pl
from jax.experimental.pallas import tpu as pltpu

_ALPHA = 0.2


def _elu(v):
    return jnp.where(v > 0, v, jnp.exp(jnp.minimum(v, 0.0)) - 1.0)


def _stgat_kernel(nhead, nhid, npred,
                  x_ref, adj_ref, wall_ref, a1_ref, a2t_ref,
                  wout_ref, a1ot_ref, a2ot_ref, out_ref,
                  adj_s, va_s, r_s, f1t_s, f2t_s, fb1_s, h_s,
                  vap_s, rp_s, f1pt_s, f2pt_s, fbp_s):
    ph = pl.program_id(1)
    bf16 = jnp.bfloat16

    @pl.when(ph == 0)
    def _layer1():
        adj_s[...] = adj_ref[0].astype(bf16)
        wh = jnp.dot(x_ref[0].astype(bf16), wall_ref[...],
                     preferred_element_type=jnp.float32)
        whb = wh.astype(bf16)
        wh1 = jnp.dot(whb, a1_ref[...],
                      preferred_element_type=jnp.float32)
        wh2t = lax.dot_general(
            a2t_ref[...], whb, (((1,), (1,)), ((), ())),
            preferred_element_type=jnp.float32)
        r_s[...] = jnp.exp((_ALPHA - 1.0) * wh1).astype(bf16)
        f1t_s[...] = jnp.exp(wh2t).astype(bf16)
        f2t_s[...] = jnp.exp(_ALPHA * wh2t).astype(bf16)
        ones_col = jnp.ones((wh.shape[0], 1), bf16)
        for h in range(nhead):
            whh = wh[:, h * nhid:(h + 1) * nhid]
            va_s[h, :, :nhid] = whb[:, h * nhid:(h + 1) * nhid]
            va_s[h, :, nhid:nhid + 1] = ones_col
            fb1_s[h:h + 1, :] = jnp.mean(whh, axis=0, keepdims=True)

        adjb = adj_s[...]
        rb = r_s[...]
        outs = []
        for h in range(nhead):
            p = jnp.maximum(f1t_s[h:h + 1, :],
                            rb[:, h:h + 1] * f2t_s[h:h + 1, :])
            w = adjb * p
            na = jnp.dot(w, va_s[h], preferred_element_type=jnp.float32)
            num = na[:, :nhid]
            den = na[:, nhid:nhid + 1]
            rden = jnp.where(den > 0, 1.0 / den, 0.0)
            hp = jnp.where(den > 0, num * rden, fb1_s[h:h + 1, :])
            outs.append(_elu(hp))
        h_s[...] = jnp.concatenate(outs, axis=1).astype(bf16)

    @pl.when(ph == 1)
    def _layer2():
        whp = jnp.dot(h_s[...], wout_ref[...],
                      preferred_element_type=jnp.float32)
        wh1p = lax.dot_general(
            whp, a1ot_ref[...], (((1,), (1,)), ((), ())),
            preferred_element_type=jnp.float32)
        wh2pt = lax.dot_general(
            a2ot_ref[...], whp, (((1,), (1,)), ((), ())),
            preferred_element_type=jnp.float32)
        rp_s[...] = jnp.exp((_ALPHA - 1.0) * wh1p).astype(bf16)
        f1pt_s[...] = jnp.exp(wh2pt).astype(bf16)
        f2pt_s[...] = jnp.exp(_ALPHA * wh2pt).astype(bf16)
        vap_s[:, :npred] = whp.astype(bf16)
        vap_s[:, npred:npred + 1] = jnp.ones((whp.shape[0], 1), bf16)
        fbp_s[...] = jnp.mean(whp, axis=0, keepdims=True)

        adjb = adj_s[...]
        p = jnp.maximum(f1pt_s[...], rp_s[...] * f2pt_s[...])
        w = adjb * p
        na = jnp.dot(w, vap_s[...], preferred_element_type=jnp.float32)
        num = na[:, :npred]
        den = na[:, npred:npred + 1]
        rden = jnp.where(den > 0, 1.0 / den, 0.0)
        hp = jnp.where(den > 0, num * rden, fbp_s[...])
        out_ref[0] = _elu(hp)


def kernel(x, adj, Ws, a_s, W_out, a_out):
    B, N, F = x.shape
    nhead, _, nhid = Ws.shape
    npred = W_out.shape[1]
    H = nhead * nhid

    wall = jnp.transpose(Ws, (1, 0, 2)).reshape(F, H).astype(jnp.bfloat16)
    eye = jnp.eye(nhead, dtype=x.dtype)
    a1 = a_s[:, :nhid, 0]
    a2 = a_s[:, nhid:, 0]
    a1bd = (eye[:, :, None] * a1[None, :, :]).reshape(nhead, H).T.astype(jnp.bfloat16)
    a2t = (eye[:, :, None] * a2[None, :, :]).reshape(nhead, H).astype(jnp.bfloat16)
    a1ot = a_out[:npred].T
    a2ot = a_out[npred:].T

    grid = (B, 2)

    def _idx_const(b, ph):
        return (0, 0)

    out = pl.pallas_call(
        lambda *refs: _stgat_kernel(nhead, nhid, npred, *refs),
        grid=grid,
        in_specs=[
            pl.BlockSpec((1, N, F), lambda b, ph: (b, 0, 0)),
            pl.BlockSpec((1, N, N), lambda b, ph: (b, 0, 0)),
            pl.BlockSpec((F, H), _idx_const),
            pl.BlockSpec((H, nhead), _idx_const),
            pl.BlockSpec((nhead, H), _idx_const),
            pl.BlockSpec((H, npred), _idx_const),
            pl.BlockSpec((1, npred), _idx_const),
            pl.BlockSpec((1, npred), _idx_const),
        ],
        out_specs=pl.BlockSpec((1, N, npred), lambda b, ph: (b, 0, 0)),
        out_shape=jax.ShapeDtypeStruct((B, N, npred), jnp.float32),
        scratch_shapes=[
            pltpu.VMEM((N, N), jnp.bfloat16),
            pltpu.VMEM((nhead, N, nhid + 1), jnp.bfloat16),
            pltpu.VMEM((N, nhead), jnp.bfloat16),
            pltpu.VMEM((nhead, N), jnp.bfloat16),
            pltpu.VMEM((nhead, N), jnp.bfloat16),
            pltpu.VMEM((nhead, nhid), jnp.float32),
            pltpu.VMEM((N, H), jnp.bfloat16),
            pltpu.VMEM((N, npred + 1), jnp.bfloat16),
            pltpu.VMEM((N, 1), jnp.bfloat16),
            pltpu.VMEM((1, N), jnp.bfloat16),
            pltpu.VMEM((1, N), jnp.bfloat16),
            pltpu.VMEM((1, npred), jnp.float32),
        ],
    )(x, adj, wall, a1bd, a2t, W_out.astype(jnp.bfloat16), a1ot, a2ot)
    return out.reshape(B, N * npred)
